# baseline (speedup 1.0000x reference)
import functools

import jax
import jax.numpy as jnp
from jax import lax
from jax.experimental import pallas as pl
from jax.experimental.pallas import tpu as pltpu
from jax.experimental.pallas import tpu_sc as plsc

_VOCAB = 100000
_D = 64
_B = 4096
_H = 50

_NC = 2
_NS = 16
_L = 16
_NW = _NC * _NS
_BPW = _B // _NW

_mesh = plsc.VectorSubcoreMesh(
    core_axis_name="c", subcore_axis_name="s", num_cores=_NC, num_subcores=_NS
)

_FW = 384
_NFULL = (_VOCAB - 160) // _FW
_C128 = _NFULL * _FW
_CTAIL = _C128 + 128


@functools.partial(
    pl.kernel,
    out_type=jax.ShapeDtypeStruct((_VOCAB // 2, 2 * _D), jnp.float32),
    mesh=_mesh,
    scratch_types=[
        pltpu.VMEM((_D, _FW), jnp.float32),
        pltpu.VMEM((_FW // 2, 2 * _D), jnp.float32),
        pltpu.VMEM((16, 128), jnp.float32),
    ],
    compiler_params=pltpu.CompilerParams(needs_layout_passes=False),
)
def _fmt_sc(tt_hbm, tail_hbm, out_hbm, buf_v, row_v, tb_v):
    wid = lax.axis_index("s") * _NC + lax.axis_index("c")
    rows0 = lax.iota(jnp.int32, 16)

    def _transpose_block(n):
        @pl.loop(0, n)
        def _rows(v):
            tv = jnp.zeros((_L,), jnp.int32) + v
            r2 = v >> 1
            c0 = (v & 1) * _D
            for q in range(_D // _L):
                vals = plsc.load_gather(buf_v, [rows0 + q * _L, tv])
                row_v[r2, pl.ds(c0 + q * _L, _L)] = vals

    for b in range(-(-_NFULL // _NW)):
        blk = wid + _NW * b

        @pl.when(blk < _NFULL)
        def _full():
            c0 = blk * _FW
            pltpu.sync_copy(tt_hbm.at[:, pl.ds(c0, _FW)], buf_v)
            _transpose_block(_FW)
            pltpu.sync_copy(row_v, out_hbm.at[pl.ds(blk * (_FW // 2), _FW // 2)])

    @pl.when(wid == 1)
    def _blk128():
        pltpu.sync_copy(
            tt_hbm.at[:, pl.ds(_C128, 128)], buf_v.at[:, pl.ds(0, 128)]
        )
        _transpose_block(128)
        pltpu.sync_copy(
            row_v.at[pl.ds(0, 64)], out_hbm.at[pl.ds(_C128 // 2, 64)]
        )

    @pl.when(wid == 0)
    def _tail():
        pltpu.sync_copy(tail_hbm, tb_v)
        pltpu.sync_copy(tb_v, out_hbm.at[pl.ds(_CTAIL // 2, 16)])


@functools.partial(
    pl.kernel,
    out_type=jax.ShapeDtypeStruct((_B, _D), jnp.float32),
    mesh=_mesh,
    scratch_types=[
        pltpu.VMEM((_BPW, _H), jnp.int32),
        pltpu.VMEM((_H, _BPW), jnp.int32),
        pltpu.VMEM((_BPW, _D), jnp.float32),
        pltpu.SemaphoreType.DMA,
    ],
    compiler_params=pltpu.CompilerParams(
        use_tc_tiling_on_sc=False, needs_layout_passes=False
    ),
)
def _cbow_sc(x_hbm, table_hbm, out_hbm, raw_v, idxt_v, acc_v, sem):
    wid = lax.axis_index("s") * _NC + lax.axis_index("c")
    base = wid * _BPW

    pltpu.sync_copy(x_hbm.at[pl.ds(base, _BPW)], raw_v)

    zeros = jnp.zeros((_L,), jnp.float32)

    @pl.loop(0, _BPW)
    def _zero(r):
        for c in range(_D // _L):
            acc_v[r, pl.ds(c * _L, _L)] = zeros

    rows0 = lax.iota(jnp.int32, 16)

    @pl.loop(0, _H)
    def _fire(t):
        tv = jnp.zeros((_L,), jnp.int32) + t
        for g in range(_BPW // _L):
            rows = rows0 + g * _L
            idxt_v[t, pl.ds(g * _L, _L)] = plsc.load_gather(raw_v, [rows, tv])
        pltpu.async_copy(table_hbm.at[idxt_v.at[t]], acc_v, sem, add=True)

    @pl.loop(0, _H)
    def _drain(t):
        pltpu.make_async_copy(table_hbm.at[idxt_v.at[0]], acc_v, sem).wait()

    inv_h = jnp.float32(1.0 / _H)

    @pl.loop(0, _BPW)
    def _scale(r):
        for c in range(_D // _L):
            sl = pl.ds(c * _L, _L)
            acc_v[r, sl] = acc_v[r, sl] * inv_h

    pltpu.sync_copy(acc_v, out_hbm.at[pl.ds(base, _BPW)])


def kernel(x, table):
    tail = table[_CTAIL:].reshape(16, 128)
    lin = _fmt_sc(table.T, tail)
    table_lin = lin.reshape(_VOCAB, _D)
    return _cbow_sc(x.astype(jnp.int32), table_lin)


# speedup vs baseline: 1.7138x; 1.7138x over previous
import functools

import jax
import jax.numpy as jnp
from jax import lax
from jax.experimental import pallas as pl
from jax.experimental.pallas import tpu as pltpu
from jax.experimental.pallas import tpu_sc as plsc

_VOCAB = 100000
_D = 64
_B = 4096
_H = 50

_NC = 2
_NS = 16
_L = 16
_NW = _NC * _NS
_BPW = _B // _NW

_mesh = plsc.VectorSubcoreMesh(
    core_axis_name="c", subcore_axis_name="s", num_cores=_NC, num_subcores=_NS
)

_FW = 384
_NFULL = (_VOCAB - 160) // _FW
_C128 = _NFULL * _FW
_CTAIL = _C128 + 128


@functools.partial(
    pl.kernel,
    out_type=jax.ShapeDtypeStruct((_VOCAB * _D,), jnp.float32),
    mesh=_mesh,
    scratch_types=[
        pltpu.VMEM((_D, _FW), jnp.float32),
        pltpu.VMEM((_FW * _D,), jnp.float32),
        pltpu.VMEM((32 * _D,), jnp.float32),
    ],
    compiler_params=pltpu.CompilerParams(needs_layout_passes=False),
)
def _fmt_sc(tt_hbm, tail_hbm, out_hbm, buf_v, rowf_v, tb_v):
    wid = lax.axis_index("s") * _NC + lax.axis_index("c")
    iota = lax.iota(jnp.int32, 16)
    rot = [(iota + j) & 15 for j in range(16)]
    dstv = [r * _D + iota for r in rot]
    dq = [iota + 16 * q for q in range(_D // _L)]

    def _transpose_block(n):
        @pl.loop(0, n, step=16)
        def _v0(v0):
            for q in range(_D // _L):
                for j in range(16):
                    g = plsc.load_gather(buf_v, [dq[q], rot[j] + v0])
                    plsc.store_scatter(
                        rowf_v, [dstv[j] + (v0 * _D + 16 * q)], g
                    )

    for b in range(-(-_NFULL // _NW)):
        blk = wid + _NW * b

        @pl.when(blk < _NFULL)
        def _full():
            c0 = blk * _FW
            pltpu.sync_copy(tt_hbm.at[:, pl.ds(c0, _FW)], buf_v)
            _transpose_block(_FW)
            pltpu.sync_copy(rowf_v, out_hbm.at[pl.ds(blk * (_FW * _D), _FW * _D)])

    @pl.when(wid == 1)
    def _blk128():
        pltpu.sync_copy(
            tt_hbm.at[:, pl.ds(_C128, 128)], buf_v.at[:, pl.ds(0, 128)]
        )
        _transpose_block(128)
        pltpu.sync_copy(
            rowf_v.at[pl.ds(0, 128 * _D)],
            out_hbm.at[pl.ds(_C128 * _D, 128 * _D)],
        )

    @pl.when(wid == 0)
    def _tail():
        pltpu.sync_copy(tail_hbm, tb_v)
        pltpu.sync_copy(tb_v, out_hbm.at[pl.ds(_CTAIL * _D, 32 * _D)])


@functools.partial(
    pl.kernel,
    out_type=jax.ShapeDtypeStruct((_B, _D), jnp.float32),
    mesh=_mesh,
    scratch_types=[
        pltpu.VMEM((_BPW, _H), jnp.int32),
        pltpu.VMEM((_H, _BPW), jnp.int32),
        pltpu.VMEM((_BPW, _D), jnp.float32),
        pltpu.SemaphoreType.DMA,
    ],
    compiler_params=pltpu.CompilerParams(
        use_tc_tiling_on_sc=False, needs_layout_passes=False
    ),
)
def _cbow_sc(x_hbm, table_hbm, out_hbm, raw_v, idxt_v, acc_v, sem):
    wid = lax.axis_index("s") * _NC + lax.axis_index("c")
    base = wid * _BPW

    pltpu.sync_copy(x_hbm.at[pl.ds(base, _BPW)], raw_v)

    zeros = jnp.zeros((_L,), jnp.float32)

    @pl.loop(0, _BPW)
    def _zero(r):
        for c in range(_D // _L):
            acc_v[r, pl.ds(c * _L, _L)] = zeros

    rows0 = lax.iota(jnp.int32, 16)

    @pl.loop(0, _H)
    def _fire(t):
        tv = jnp.zeros((_L,), jnp.int32) + t
        for g in range(_BPW // _L):
            rows = rows0 + g * _L
            idxt_v[t, pl.ds(g * _L, _L)] = plsc.load_gather(raw_v, [rows, tv])
        pltpu.async_copy(table_hbm.at[idxt_v.at[t]], acc_v, sem, add=True)

    @pl.loop(0, _H)
    def _drain(t):
        pltpu.make_async_copy(table_hbm.at[idxt_v.at[0]], acc_v, sem).wait()

    inv_h = jnp.float32(1.0 / _H)

    @pl.loop(0, _BPW)
    def _scale(r):
        for c in range(_D // _L):
            sl = pl.ds(c * _L, _L)
            acc_v[r, sl] = acc_v[r, sl] * inv_h

    pltpu.sync_copy(acc_v, out_hbm.at[pl.ds(base, _BPW)])


def kernel(x, table):
    tail = table[_CTAIL:].reshape(-1)
    lin = _fmt_sc(table.T, tail)
    table_lin = lin.reshape(_VOCAB, _D)
    return _cbow_sc(x.astype(jnp.int32), table_lin)


# speedup vs baseline: 1.7939x; 1.0467x over previous
import functools

import jax
import jax.numpy as jnp
from jax import lax
from jax.experimental import pallas as pl
from jax.experimental.pallas import tpu as pltpu
from jax.experimental.pallas import tpu_sc as plsc

_VOCAB = 100000
_D = 64
_B = 4096
_H = 50

_NC = 2
_NS = 16
_L = 16
_NW = _NC * _NS
_BPW = _B // _NW

_RS = 72

_mesh = plsc.VectorSubcoreMesh(
    core_axis_name="c", subcore_axis_name="s", num_cores=_NC, num_subcores=_NS
)

_FW = 384
_NFULL = (_VOCAB - 160) // _FW
_C128 = _NFULL * _FW
_CTAIL = _C128 + 128


@functools.partial(
    pl.kernel,
    out_type=jax.ShapeDtypeStruct((_VOCAB * _RS,), jnp.float32),
    mesh=_mesh,
    scratch_types=[
        pltpu.VMEM((_D, _FW), jnp.float32),
        pltpu.VMEM((_FW * _RS,), jnp.float32),
        pltpu.VMEM((32 * _D,), jnp.float32),
        pltpu.VMEM((32 * _RS,), jnp.float32),
    ],
    compiler_params=pltpu.CompilerParams(needs_layout_passes=False),
)
def _fmt_sc(tt_hbm, tail_hbm, out_hbm, buf_v, rowf_v, tb_v, tb2_v):
    wid = lax.axis_index("s") * _NC + lax.axis_index("c")
    iota = lax.iota(jnp.int32, 16)
    iota_rs = iota * _RS

    def _transpose_block(n):
        @pl.loop(0, n, step=16)
        def _v0(v0):
            for d in range(_D):
                g = buf_v[d, pl.ds(v0, _L)]
                plsc.store_scatter(rowf_v, [iota_rs + (v0 * _RS + d)], g)

    for b in range(-(-_NFULL // _NW)):
        blk = wid + _NW * b

        @pl.when(blk < _NFULL)
        def _full():
            c0 = blk * _FW
            pltpu.sync_copy(tt_hbm.at[:, pl.ds(c0, _FW)], buf_v)
            _transpose_block(_FW)
            pltpu.sync_copy(
                rowf_v, out_hbm.at[pl.ds(blk * (_FW * _RS), _FW * _RS)]
            )

    @pl.when(wid == 1)
    def _blk128():
        pltpu.sync_copy(
            tt_hbm.at[:, pl.ds(_C128, 128)], buf_v.at[:, pl.ds(0, 128)]
        )
        _transpose_block(128)
        pltpu.sync_copy(
            rowf_v.at[pl.ds(0, 128 * _RS)],
            out_hbm.at[pl.ds(_C128 * _RS, 128 * _RS)],
        )

    @pl.when(wid == 0)
    def _tail():
        pltpu.sync_copy(tail_hbm, tb_v)

        @pl.loop(0, 32)
        def _r(r):
            for q in range(_D // _L):
                tb2_v[pl.ds(r * _RS + q * _L, _L)] = tb_v[
                    pl.ds(r * _D + q * _L, _L)
                ]

        pltpu.sync_copy(tb2_v, out_hbm.at[pl.ds(_CTAIL * _RS, 32 * _RS)])


@functools.partial(
    pl.kernel,
    out_type=jax.ShapeDtypeStruct((_B, _D), jnp.float32),
    mesh=_mesh,
    scratch_types=[
        pltpu.VMEM((_BPW, _H), jnp.int32),
        pltpu.VMEM((_H, _BPW), jnp.int32),
        pltpu.VMEM((_BPW, _RS), jnp.float32),
        pltpu.SemaphoreType.DMA,
    ],
    compiler_params=pltpu.CompilerParams(
        use_tc_tiling_on_sc=False, needs_layout_passes=False
    ),
)
def _cbow_sc(x_hbm, table_hbm, out_hbm, raw_v, idxt_v, acc_v, sem):
    wid = lax.axis_index("s") * _NC + lax.axis_index("c")
    base = wid * _BPW

    pltpu.sync_copy(x_hbm.at[pl.ds(base, _BPW)], raw_v)

    zeros = jnp.zeros((_L,), jnp.float32)

    @pl.loop(0, _BPW)
    def _zero_rows(r):
        for c in range(_D // _L):
            acc_v[r, pl.ds(c * _L, _L)] = zeros
        acc_v[r, pl.ds(_RS - _L, _L)] = zeros

    rows0 = lax.iota(jnp.int32, 16)

    @pl.loop(0, _H)
    def _fire(t):
        tv = jnp.zeros((_L,), jnp.int32) + t
        for g in range(_BPW // _L):
            rows = rows0 + g * _L
            idxt_v[t, pl.ds(g * _L, _L)] = plsc.load_gather(raw_v, [rows, tv])
        pltpu.async_copy(table_hbm.at[idxt_v.at[t]], acc_v, sem, add=True)

    @pl.loop(0, _H)
    def _drain(t):
        pltpu.make_async_copy(table_hbm.at[idxt_v.at[0]], acc_v, sem).wait()

    inv_h = jnp.float32(1.0 / _H)

    @pl.loop(0, _BPW)
    def _scale(r):
        for c in range(_D // _L):
            sl = pl.ds(c * _L, _L)
            acc_v[r, sl] = acc_v[r, sl] * inv_h

    pltpu.sync_copy(acc_v.at[:, pl.ds(0, _D)], out_hbm.at[pl.ds(base, _BPW)])


def kernel(x, table):
    tail = table[_CTAIL:].reshape(-1)
    lin = _fmt_sc(table.T, tail)
    table_rs = lin.reshape(_VOCAB, _RS)
    return _cbow_sc(x.astype(jnp.int32), table_rs)


# speedup vs baseline: 2.5122x; 1.4004x over previous
import functools

import jax
import jax.numpy as jnp
from jax import lax
from jax.experimental import pallas as pl
from jax.experimental.pallas import tpu as pltpu
from jax.experimental.pallas import tpu_sc as plsc

_VOCAB = 100000
_D = 64
_B = 4096
_H = 50

_NC = 2
_NS = 16
_L = 16
_NW = _NC * _NS
_BPW = _B // _NW

_RS = 72

_mesh = plsc.VectorSubcoreMesh(
    core_axis_name="c", subcore_axis_name="s", num_cores=_NC, num_subcores=_NS
)

_FW = 384
_NFULL = (_VOCAB - 160) // _FW
_C128 = _NFULL * _FW
_CTAIL = _C128 + 128


@functools.partial(
    pl.kernel,
    out_type=jax.ShapeDtypeStruct((_VOCAB * _RS,), jnp.float32),
    mesh=_mesh,
    scratch_types=[
        pltpu.VMEM((_D, _FW), jnp.float32),
        pltpu.VMEM((_FW * _RS,), jnp.float32),
        pltpu.VMEM((32 * _D,), jnp.float32),
        pltpu.VMEM((32 * _RS,), jnp.float32),
    ],
    compiler_params=pltpu.CompilerParams(needs_layout_passes=False),
)
def _fmt_sc(tt_hbm, tail_hbm, out_hbm, buf_v, rowf_v, tb_v, tb2_v):
    wid = lax.axis_index("s") * _NC + lax.axis_index("c")
    iota = lax.iota(jnp.int32, 16)
    iota_rs = iota * _RS

    def _transpose_block(n):
        @pl.loop(0, n, step=16)
        def _v0(v0):
            for d0 in range(0, _D, 8):
                gs = [buf_v[d0 + i, pl.ds(v0, _L)] for i in range(8)]
                for i in range(8):
                    plsc.store_scatter(
                        rowf_v, [iota_rs + (v0 * _RS + d0 + i)], gs[i]
                    )

    for b in range(-(-_NFULL // _NW)):
        blk = wid + _NW * b

        @pl.when(blk < _NFULL)
        def _full():
            c0 = blk * _FW
            pltpu.sync_copy(tt_hbm.at[:, pl.ds(c0, _FW)], buf_v)
            _transpose_block(_FW)
            pltpu.sync_copy(
                rowf_v, out_hbm.at[pl.ds(blk * (_FW * _RS), _FW * _RS)]
            )

    @pl.when(wid == 1)
    def _blk128():
        pltpu.sync_copy(
            tt_hbm.at[:, pl.ds(_C128, 128)], buf_v.at[:, pl.ds(0, 128)]
        )
        _transpose_block(128)
        pltpu.sync_copy(
            rowf_v.at[pl.ds(0, 128 * _RS)],
            out_hbm.at[pl.ds(_C128 * _RS, 128 * _RS)],
        )

    @pl.when(wid == 0)
    def _tail():
        pltpu.sync_copy(tail_hbm, tb_v)

        @pl.loop(0, 32)
        def _r(r):
            for q in range(_D // _L):
                tb2_v[pl.ds(r * _RS + q * _L, _L)] = tb_v[
                    pl.ds(r * _D + q * _L, _L)
                ]

        pltpu.sync_copy(tb2_v, out_hbm.at[pl.ds(_CTAIL * _RS, 32 * _RS)])


@functools.partial(
    pl.kernel,
    out_type=jax.ShapeDtypeStruct((_B, _D), jnp.float32),
    mesh=_mesh,
    scratch_types=[
        pltpu.VMEM((_BPW, _H), jnp.int32),
        pltpu.VMEM((_H, _BPW), jnp.int32),
        pltpu.VMEM((_BPW, _RS), jnp.float32),
        pltpu.SemaphoreType.DMA,
    ],
    compiler_params=pltpu.CompilerParams(
        use_tc_tiling_on_sc=False, needs_layout_passes=False
    ),
)
def _cbow_sc(x_hbm, table_hbm, out_hbm, raw_v, idxt_v, acc_v, sem):
    wid = lax.axis_index("s") * _NC + lax.axis_index("c")
    base = wid * _BPW

    pltpu.sync_copy(x_hbm.at[pl.ds(base, _BPW)], raw_v)

    zeros = jnp.zeros((_L,), jnp.float32)

    @pl.loop(0, _BPW)
    def _zero_rows(r):
        for c in range(_D // _L):
            acc_v[r, pl.ds(c * _L, _L)] = zeros
        acc_v[r, pl.ds(_RS - _L, _L)] = zeros

    rows0 = lax.iota(jnp.int32, 16)

    @pl.loop(0, _H)
    def _fire(t):
        tv = jnp.zeros((_L,), jnp.int32) + t
        for g in range(_BPW // _L):
            rows = rows0 + g * _L
            idxt_v[t, pl.ds(g * _L, _L)] = plsc.load_gather(raw_v, [rows, tv])
        pltpu.async_copy(table_hbm.at[idxt_v.at[t]], acc_v, sem, add=True)

    @pl.loop(0, _H)
    def _drain(t):
        pltpu.make_async_copy(table_hbm.at[idxt_v.at[0]], acc_v, sem).wait()

    inv_h = jnp.float32(1.0 / _H)

    @pl.loop(0, _BPW)
    def _scale(r):
        for c in range(_D // _L):
            sl = pl.ds(c * _L, _L)
            acc_v[r, sl] = acc_v[r, sl] * inv_h

    pltpu.sync_copy(acc_v.at[:, pl.ds(0, _D)], out_hbm.at[pl.ds(base, _BPW)])


def kernel(x, table):
    tail = table[_CTAIL:].reshape(-1)
    lin = _fmt_sc(table.T, tail)
    table_rs = lin.reshape(_VOCAB, _RS)
    return _cbow_sc(x.astype(jnp.int32), table_rs)


# speedup vs baseline: 3.0539x; 1.2156x over previous
import functools

import jax
import jax.numpy as jnp
from jax import lax
from jax.experimental import pallas as pl
from jax.experimental.pallas import tpu as pltpu
from jax.experimental.pallas import tpu_sc as plsc

_VOCAB = 100000
_D = 64
_B = 4096
_H = 50

_NC = 2
_NS = 16
_L = 16
_NW = _NC * _NS
_BPW = _B // _NW

_RS = 72

_mesh = plsc.VectorSubcoreMesh(
    core_axis_name="c", subcore_axis_name="s", num_cores=_NC, num_subcores=_NS
)

_FW = 384
_NFULL = (_VOCAB - 160) // _FW
_C128 = _NFULL * _FW
_CTAIL = _C128 + 128


_NUNIF = (_NFULL // _NW) * _NW


@functools.partial(
    pl.kernel,
    out_type=jax.ShapeDtypeStruct((_VOCAB * _RS,), jnp.float32),
    mesh=_mesh,
    scratch_types=[
        pltpu.VMEM((_D, _FW), jnp.float32),
        pltpu.VMEM((_D, _FW), jnp.float32),
        pltpu.VMEM((_FW * _RS,), jnp.float32),
        pltpu.VMEM((_FW * _RS,), jnp.float32),
        pltpu.VMEM((32 * _D,), jnp.float32),
        pltpu.VMEM((32 * _RS,), jnp.float32),
        pltpu.SemaphoreType.DMA,
        pltpu.SemaphoreType.DMA,
    ],
    compiler_params=pltpu.CompilerParams(needs_layout_passes=False),
)
def _fmt_sc(tt_hbm, tail_hbm, out_hbm, buf0, buf1, rf0, rf1, tb_v, tb2_v,
            sin, sout):
    wid = lax.axis_index("s") * _NC + lax.axis_index("c")
    iota = lax.iota(jnp.int32, 16)
    iota_rs = iota * _RS
    bufs = (buf0, buf1)
    rfs = (rf0, rf1)

    def _transpose_block(buf_v, rowf_v, n):
        @pl.loop(0, n, step=16)
        def _v0(v0):
            for d0 in range(0, _D, 8):
                gs = [buf_v[d0 + i, pl.ds(v0, _L)] for i in range(8)]
                for i in range(8):
                    plsc.store_scatter(
                        rowf_v, [iota_rs + (v0 * _RS + d0 + i)], gs[i]
                    )

    def _in_copy(b, buf_v):
        c0 = (wid + _NW * b) * _FW
        return pltpu.make_async_copy(tt_hbm.at[:, pl.ds(c0, _FW)], buf_v, sin)

    def _out_copy(b, rowf_v):
        o0 = (wid + _NW * b) * (_FW * _RS)
        return pltpu.make_async_copy(
            rowf_v, out_hbm.at[pl.ds(o0, _FW * _RS)], sout
        )

    nrounds = _NUNIF // _NW
    _in_copy(0, bufs[0]).start()
    for b in range(nrounds):
        _in_copy(b, bufs[b % 2]).wait()
        if b + 1 < nrounds:
            _in_copy(b + 1, bufs[(b + 1) % 2]).start()
        if b >= 2:
            _out_copy(b - 2, rfs[b % 2]).wait()
        _transpose_block(bufs[b % 2], rfs[b % 2], _FW)
        _out_copy(b, rfs[b % 2]).start()
    _out_copy(nrounds - 2, rfs[nrounds % 2]).wait()
    _out_copy(nrounds - 1, rfs[(nrounds - 1) % 2]).wait()

    @pl.when(wid < _NFULL - _NUNIF)
    def _rem():
        blk = _NUNIF + wid
        c0 = blk * _FW
        pltpu.sync_copy(tt_hbm.at[:, pl.ds(c0, _FW)], buf0)
        _transpose_block(buf0, rf0, _FW)
        pltpu.sync_copy(rf0, out_hbm.at[pl.ds(blk * (_FW * _RS), _FW * _RS)])

    @pl.when(wid == 4)
    def _blk128():
        pltpu.sync_copy(
            tt_hbm.at[:, pl.ds(_C128, 128)], buf0.at[:, pl.ds(0, 128)]
        )
        _transpose_block(buf0, rf0, 128)
        pltpu.sync_copy(
            rf0.at[pl.ds(0, 128 * _RS)],
            out_hbm.at[pl.ds(_C128 * _RS, 128 * _RS)],
        )

    @pl.when(wid == 5)
    def _tail():
        pltpu.sync_copy(tail_hbm, tb_v)

        @pl.loop(0, 32)
        def _r(r):
            for q in range(_D // _L):
                tb2_v[pl.ds(r * _RS + q * _L, _L)] = tb_v[
                    pl.ds(r * _D + q * _L, _L)
                ]

        pltpu.sync_copy(tb2_v, out_hbm.at[pl.ds(_CTAIL * _RS, 32 * _RS)])


@functools.partial(
    pl.kernel,
    out_type=jax.ShapeDtypeStruct((_B, _D), jnp.float32),
    mesh=_mesh,
    scratch_types=[
        pltpu.VMEM((_BPW, _H), jnp.int32),
        pltpu.VMEM((_H, _BPW), jnp.int32),
        pltpu.VMEM((_BPW, _RS), jnp.float32),
        pltpu.SemaphoreType.DMA,
    ],
    compiler_params=pltpu.CompilerParams(
        use_tc_tiling_on_sc=False, needs_layout_passes=False
    ),
)
def _cbow_sc(x_hbm, table_hbm, out_hbm, raw_v, idxt_v, acc_v, sem):
    wid = lax.axis_index("s") * _NC + lax.axis_index("c")
    base = wid * _BPW

    pltpu.sync_copy(x_hbm.at[pl.ds(base, _BPW)], raw_v)

    zeros = jnp.zeros((_L,), jnp.float32)

    @pl.loop(0, _BPW)
    def _zero_rows(r):
        for c in range(_D // _L):
            acc_v[r, pl.ds(c * _L, _L)] = zeros
        acc_v[r, pl.ds(_RS - _L, _L)] = zeros

    rows0 = lax.iota(jnp.int32, 16)

    @pl.loop(0, _H)
    def _fire(t):
        tv = jnp.zeros((_L,), jnp.int32) + t
        for g in range(_BPW // _L):
            rows = rows0 + g * _L
            idxt_v[t, pl.ds(g * _L, _L)] = plsc.load_gather(raw_v, [rows, tv])
        pltpu.async_copy(table_hbm.at[idxt_v.at[t]], acc_v, sem, add=True)

    @pl.loop(0, _H)
    def _drain(t):
        pltpu.make_async_copy(table_hbm.at[idxt_v.at[0]], acc_v, sem).wait()

    inv_h = jnp.float32(1.0 / _H)

    @pl.loop(0, _BPW)
    def _scale(r):
        for c in range(_D // _L):
            sl = pl.ds(c * _L, _L)
            acc_v[r, sl] = acc_v[r, sl] * inv_h

    pltpu.sync_copy(acc_v.at[:, pl.ds(0, _D)], out_hbm.at[pl.ds(base, _BPW)])


def kernel(x, table):
    tail = table[_CTAIL:].reshape(-1)
    lin = _fmt_sc(table.T, tail)
    table_rs = lin.reshape(_VOCAB, _RS)
    return _cbow_sc(x.astype(jnp.int32), table_rs)


# speedup vs baseline: 3.0745x; 1.0067x over previous
import functools

import jax
import jax.numpy as jnp
from jax import lax
from jax.experimental import pallas as pl
from jax.experimental.pallas import tpu as pltpu
from jax.experimental.pallas import tpu_sc as plsc

_VOCAB = 100000
_D = 64
_B = 4096
_H = 50

_NC = 2
_NS = 16
_L = 16
_NW = _NC * _NS
_BPW = _B // _NW

_RS = 72

_mesh = plsc.VectorSubcoreMesh(
    core_axis_name="c", subcore_axis_name="s", num_cores=_NC, num_subcores=_NS
)

_FW = 384
_NFULL = (_VOCAB - 160) // _FW
_C128 = _NFULL * _FW
_CTAIL = _C128 + 128


_NUNIF = (_NFULL // _NW) * _NW


@functools.partial(
    pl.kernel,
    out_type=jax.ShapeDtypeStruct((_VOCAB * _RS,), jnp.float32),
    mesh=_mesh,
    scratch_types=[
        pltpu.VMEM((_D, _FW), jnp.float32),
        pltpu.VMEM((_D, _FW), jnp.float32),
        pltpu.VMEM((_FW * _RS,), jnp.float32),
        pltpu.VMEM((_FW * _RS,), jnp.float32),
        pltpu.VMEM((32 * _D,), jnp.float32),
        pltpu.VMEM((32 * _RS,), jnp.float32),
        pltpu.SemaphoreType.DMA,
        pltpu.SemaphoreType.DMA,
    ],
    compiler_params=pltpu.CompilerParams(needs_layout_passes=False),
)
def _fmt_sc(tt_hbm, tail_hbm, out_hbm, buf0, buf1, rf0, rf1, tb_v, tb2_v,
            sin, sout):
    wid = lax.axis_index("s") * _NC + lax.axis_index("c")
    iota = lax.iota(jnp.int32, 16)
    iota_rs = iota * _RS
    bufs = (buf0, buf1)
    rfs = (rf0, rf1)

    def _transpose_block(buf_v, rowf_v, n):
        @pl.loop(0, n, step=16)
        def _v0(v0):
            for d0 in range(0, _D, 8):
                gs = [buf_v[d0 + i, pl.ds(v0, _L)] for i in range(8)]
                for i in range(8):
                    plsc.store_scatter(
                        rowf_v, [iota_rs + (v0 * _RS + d0 + i)], gs[i]
                    )

    def _in_copy(b, buf_v):
        c0 = (wid + _NW * b) * _FW
        return pltpu.make_async_copy(tt_hbm.at[:, pl.ds(c0, _FW)], buf_v, sin)

    def _out_copy(b, rowf_v):
        o0 = (wid + _NW * b) * (_FW * _RS)
        return pltpu.make_async_copy(
            rowf_v, out_hbm.at[pl.ds(o0, _FW * _RS)], sout
        )

    nrounds = _NUNIF // _NW
    _in_copy(0, bufs[0]).start()
    for b in range(nrounds):
        _in_copy(b, bufs[b % 2]).wait()
        if b + 1 < nrounds:
            _in_copy(b + 1, bufs[(b + 1) % 2]).start()
        if b >= 2:
            _out_copy(b - 2, rfs[b % 2]).wait()
        _transpose_block(bufs[b % 2], rfs[b % 2], _FW)
        _out_copy(b, rfs[b % 2]).start()
    _out_copy(nrounds - 2, rfs[nrounds % 2]).wait()
    _out_copy(nrounds - 1, rfs[(nrounds - 1) % 2]).wait()

    @pl.when(wid < _NFULL - _NUNIF)
    def _rem():
        blk = _NUNIF + wid
        c0 = blk * _FW
        pltpu.sync_copy(tt_hbm.at[:, pl.ds(c0, _FW)], buf0)
        _transpose_block(buf0, rf0, _FW)
        pltpu.sync_copy(rf0, out_hbm.at[pl.ds(blk * (_FW * _RS), _FW * _RS)])

    @pl.when(wid == 4)
    def _blk128():
        pltpu.sync_copy(
            tt_hbm.at[:, pl.ds(_C128, 128)], buf0.at[:, pl.ds(0, 128)]
        )
        _transpose_block(buf0, rf0, 128)
        pltpu.sync_copy(
            rf0.at[pl.ds(0, 128 * _RS)],
            out_hbm.at[pl.ds(_C128 * _RS, 128 * _RS)],
        )

    @pl.when(wid == 5)
    def _tail():
        pltpu.sync_copy(tail_hbm, tb_v)

        @pl.loop(0, 32)
        def _r(r):
            for q in range(_D // _L):
                tb2_v[pl.ds(r * _RS + q * _L, _L)] = tb_v[
                    pl.ds(r * _D + q * _L, _L)
                ]

        pltpu.sync_copy(tb2_v, out_hbm.at[pl.ds(_CTAIL * _RS, 32 * _RS)])


@functools.partial(
    pl.kernel,
    out_type=jax.ShapeDtypeStruct((_B, _D), jnp.float32),
    mesh=_mesh,
    scratch_types=[
        pltpu.VMEM((_BPW, _H), jnp.int32),
        pltpu.VMEM((_H, _BPW), jnp.int32),
        pltpu.VMEM((_BPW, _RS), jnp.float32),
        pltpu.SemaphoreType.DMA,
    ],
    compiler_params=pltpu.CompilerParams(
        use_tc_tiling_on_sc=False, needs_layout_passes=False
    ),
)
def _cbow_sc(x_hbm, table_hbm, out_hbm, raw_v, idxt_v, acc_v, sem):
    wid = lax.axis_index("s") * _NC + lax.axis_index("c")
    base = wid * _BPW

    pltpu.sync_copy(x_hbm.at[pl.ds(base, _BPW)], raw_v)

    zeros = jnp.zeros((_L,), jnp.float32)

    @pl.loop(0, _BPW)
    def _zero_rows(r):
        for c in range(_D // _L):
            acc_v[r, pl.ds(c * _L, _L)] = zeros
        acc_v[r, pl.ds(_RS - _L, _L)] = zeros

    rows0 = lax.iota(jnp.int32, 16)

    @pl.loop(0, _H)
    def _fire(t):
        tv = jnp.zeros((_L,), jnp.int32) + t
        gs = [
            plsc.load_gather(raw_v, [rows0 + g * _L, tv])
            for g in range(_BPW // _L)
        ]
        for g in range(_BPW // _L):
            idxt_v[t, pl.ds(g * _L, _L)] = gs[g]
        pltpu.async_copy(table_hbm.at[idxt_v.at[t]], acc_v, sem, add=True)

    @pl.loop(0, _H)
    def _drain(t):
        pltpu.make_async_copy(table_hbm.at[idxt_v.at[0]], acc_v, sem).wait()

    inv_h = jnp.float32(1.0 / _H)

    @pl.loop(0, _BPW, step=2)
    def _scale(r):
        sls = [(r + i // 4, pl.ds((i % 4) * _L, _L)) for i in range(8)]
        vs = [acc_v[rr, sl] * inv_h for rr, sl in sls]
        for (rr, sl), v in zip(sls, vs):
            acc_v[rr, sl] = v

    pltpu.sync_copy(acc_v.at[:, pl.ds(0, _D)], out_hbm.at[pl.ds(base, _BPW)])


def kernel(x, table):
    tail = table[_CTAIL:].reshape(-1)
    lin = _fmt_sc(table.T, tail)
    table_rs = lin.reshape(_VOCAB, _RS)
    return _cbow_sc(x.astype(jnp.int32), table_rs)
